# baseline (speedup 1.0000x reference)
import jax
import jax.numpy as jnp
from jax import lax
from jax.experimental import pallas as pl
from jax.experimental.pallas import tpu as pltpu
from jax.experimental.pallas import tpu_sc as plsc

N = 10000
E = 320000
D = 128
R2 = 16
NB = 4

TB = 1000
NBLK = N // TB

N_PAD = 10240
NCORES = 2
NSUB = 16
NW = NCORES * NSUB
CB = 64
CEW = 320
NBUF = 4
E_PAD = NSUB * CEW * CB
ROWS_PER_SUB = N_PAD // NSUB


def _pre_body(wc_ref, w_ref, x_ref, re_ref, wrw_ref, wrb_ref, y_ref, reo_ref):
    i = pl.program_id(0)
    r = pl.program_id(1)
    rw = (wc_ref[r, 0] * w_ref[0] + wc_ref[r, 1] * w_ref[1]
          + wc_ref[r, 2] * w_ref[2] + wc_ref[r, 3] * w_ref[3])
    y_ref[...] = jnp.dot(x_ref[...], rw, preferred_element_type=jnp.float32)

    @pl.when((i == 0) & (r == 0))
    def _():
        reo_ref[...] = lax.dot_general(
            re_ref[...], wrw_ref[...], (((1,), (1,)), ((), ())),
            preferred_element_type=jnp.float32) + wrb_ref[...]


def _sc_body(y_ref, idx_ref, acc0_ref, acc1_ref, deg0_ref, deg1_ref,
             ib0, ib1, ib2, ib3, rows0, rows1, rows2, rows3,
             ones_v, zflat_v, acc_sh, deg_sh,
             isem0, isem1, isem2, isem3, gsem0, gsem1, gsem2, gsem3):
    cid = lax.axis_index("c")
    sid = lax.axis_index("s")
    wid = cid * NSUB + sid
    base = sid * ROWS_PER_SUB
    sets = ((ib0, rows0, isem0, gsem0), (ib1, rows1, isem1, gsem1),
            (ib2, rows2, isem2, gsem2), (ib3, rows3, isem3, gsem3))

    zvec = jnp.zeros((16,), jnp.float32)
    onev = jnp.ones((16,), jnp.float32)
    for g in range(CB // 16):
        ones_v[pl.ds(g * 16, 16)] = onev

    def zfill(rr, _):
        for g in range(D // 16):
            rows0[rr, pl.ds(g * 16, 16)] = zvec
        return 0

    lax.fori_loop(0, CB, zfill, 0)

    def zflat(k, _):
        zflat_v[pl.ds(k * 16, 16)] = zvec
        return 0

    lax.fori_loop(0, ROWS_PER_SUB // 16, zflat, 0)

    for t in range(ROWS_PER_SUB // CB):
        pltpu.sync_copy(rows0, acc_sh.at[pl.ds(base + t * CB, CB)])
    pltpu.sync_copy(zflat_v, deg_sh.at[pl.ds(base, ROWS_PER_SUB)])
    plsc.subcore_barrier()

    def _idx_start(j, s):
        ib, _, isem, _ = sets[s]
        pltpu.make_async_copy(idx_ref.at[sid * CEW + j], ib, isem).start()

    def _idx_wait(s):
        ib, _, isem, _ = sets[s]
        pltpu.make_async_copy(idx_ref.at[wid], ib, isem).wait()

    def _gather_start(s):
        ib, rows, _, gsem = sets[s]
        pltpu.make_async_copy(y_ref.at[ib.at[0]], rows, gsem).start()

    def _gather_wait_scatter(s):
        ib, rows, _, gsem = sets[s]
        pltpu.make_async_copy(y_ref.at[ib.at[0]], rows, gsem).wait()
        pltpu.sync_copy(rows, acc_sh.at[ib.at[1]], add=True)
        pltpu.sync_copy(ones_v, deg_sh.at[ib.at[1]], add=True)

    @pl.when(cid == 0)
    def _():
        for t in range(NBUF):
            _idx_start(t, t)
        for t in range(NBUF - 1):
            _idx_wait(t)
            _gather_start(t)

        def body(k, _):
            for t in range(NBUF):
                j = k * NBUF + t
                s_next = (t + NBUF - 1) % NBUF

                @pl.when(j + NBUF - 1 < CEW)
                def _():
                    _idx_wait(s_next)
                    _gather_start(s_next)

                _gather_wait_scatter(t)

                @pl.when(j + NBUF < CEW)
                def _():
                    _idx_start(j + NBUF, t)
            return 0

        lax.fori_loop(0, CEW // NBUF, body, 0)

    plsc.subcore_barrier()

    @pl.when(cid == 0)
    def _():
        pltpu.sync_copy(acc_sh.at[pl.ds(base, ROWS_PER_SUB)],
                        acc0_ref.at[pl.ds(base, ROWS_PER_SUB)])
        pltpu.sync_copy(deg_sh.at[pl.ds(base, ROWS_PER_SUB)],
                        deg0_ref.at[pl.ds(base, ROWS_PER_SUB)])

    @pl.when(cid == 1)
    def _():
        pltpu.sync_copy(acc_sh.at[pl.ds(base, ROWS_PER_SUB)],
                        acc1_ref.at[pl.ds(base, ROWS_PER_SUB)])
        pltpu.sync_copy(deg_sh.at[pl.ds(base, ROWS_PER_SUB)],
                        deg1_ref.at[pl.ds(base, ROWS_PER_SUB)])


def _post_body(acc0_ref, acc1_ref, dcol_ref, x_ref, w_ref, out_ref):
    a = acc0_ref[...] + acc1_ref[...]
    alpha = 1.0 / jnp.maximum(dcol_ref[...], 1.0)
    xb = x_ref[...]
    h = a * alpha + jnp.dot(xb, w_ref[...], preferred_element_type=jnp.float32)
    out_ref[:, :D] = xb
    out_ref[:, D:] = h


def kernel(x, edge_index, edge_type, rel_emb, weight, w_comp,
           self_loop_weight, W_R_w, W_R_b):
    src = edge_index[0]
    dst = edge_index[1]
    gidx = edge_type.astype(jnp.int32) * N + src
    gidx_p = jnp.pad(gidx, (0, E_PAD - E)).reshape(NSUB * CEW, 1, CB)
    dst_p = jnp.pad(dst, (0, E_PAD - E),
                    constant_values=N).reshape(NSUB * CEW, 1, CB)
    idx2 = jnp.concatenate([gidx_p, dst_p], axis=1)

    y, rel_emb_new = pl.pallas_call(
        _pre_body,
        grid=(NBLK, R2),
        in_specs=[
            pl.BlockSpec(memory_space=pltpu.SMEM),
            pl.BlockSpec((NB, D, D), lambda i, r: (0, 0, 0)),
            pl.BlockSpec((TB, D), lambda i, r: (i, 0)),
            pl.BlockSpec((R2, D), lambda i, r: (0, 0)),
            pl.BlockSpec((D, D), lambda i, r: (0, 0)),
            pl.BlockSpec((1, D), lambda i, r: (0, 0)),
        ],
        out_specs=[
            pl.BlockSpec((TB, D), lambda i, r: (r * NBLK + i, 0)),
            pl.BlockSpec((R2, D), lambda i, r: (0, 0)),
        ],
        out_shape=[
            jax.ShapeDtypeStruct((R2 * N, D), jnp.float32),
            jax.ShapeDtypeStruct((R2, D), jnp.float32),
        ],
    )(w_comp, weight, x, rel_emb, W_R_w, W_R_b.reshape(1, D))

    mesh = plsc.VectorSubcoreMesh(core_axis_name="c", subcore_axis_name="s",
                                  num_cores=NCORES, num_subcores=NSUB)
    acc0, acc1, deg0, deg1 = pl.kernel(
        _sc_body,
        out_type=[jax.ShapeDtypeStruct((N_PAD, D), jnp.float32),
                  jax.ShapeDtypeStruct((N_PAD, D), jnp.float32),
                  jax.ShapeDtypeStruct((N_PAD,), jnp.float32),
                  jax.ShapeDtypeStruct((N_PAD,), jnp.float32)],
        mesh=mesh,
        compiler_params=pltpu.CompilerParams(use_tc_tiling_on_sc=False),
        scratch_types=(
            [pltpu.VMEM((2, CB), jnp.int32) for _ in range(NBUF)]
            + [pltpu.VMEM((CB, D), jnp.float32) for _ in range(NBUF)]
            + [pltpu.VMEM((CB,), jnp.float32),
               pltpu.VMEM((ROWS_PER_SUB,), jnp.float32),
               pltpu.VMEM_SHARED((N_PAD, D), jnp.float32),
               pltpu.VMEM_SHARED((N_PAD,), jnp.float32)]
            + [pltpu.SemaphoreType.DMA for _ in range(2 * NBUF)]
        ),
    )(y, idx2)

    dcol = (deg0 + deg1).reshape(N_PAD, 1)

    repr_ = pl.pallas_call(
        _post_body,
        grid=(NBLK,),
        in_specs=[
            pl.BlockSpec((TB, D), lambda i: (i, 0)),
            pl.BlockSpec((TB, D), lambda i: (i, 0)),
            pl.BlockSpec((TB, 1), lambda i: (i, 0)),
            pl.BlockSpec((TB, D), lambda i: (i, 0)),
            pl.BlockSpec((D, D), lambda i: (0, 0)),
        ],
        out_specs=pl.BlockSpec((TB, 2 * D), lambda i: (i, 0)),
        out_shape=jax.ShapeDtypeStruct((N, 2 * D), jnp.float32),
    )(acc0, acc1, dcol, x, self_loop_weight)

    return rel_emb_new, repr_


# speedup vs baseline: 1.2815x; 1.2815x over previous
import jax
import jax.numpy as jnp
from jax import lax
from jax.experimental import pallas as pl
from jax.experimental.pallas import tpu as pltpu
from jax.experimental.pallas import tpu_sc as plsc

N = 10000
E = 320000
D = 128
R2 = 16
NB = 4

TB = 1000
NBLK = N // TB

N_PAD = 10240
NCORES = 2
NSUB = 16
NW = NCORES * NSUB
CB = 128
CE = 80
NBUF = 2
E_PAD = NW * CE * CB
ROWS_PER_SUB = N_PAD // NSUB


def _pre_body(wc_ref, w_ref, x_ref, re_ref, wrw_ref, wrb_ref, y_ref, reo_ref):
    i = pl.program_id(0)
    r = pl.program_id(1)
    rw = (wc_ref[r, 0] * w_ref[0] + wc_ref[r, 1] * w_ref[1]
          + wc_ref[r, 2] * w_ref[2] + wc_ref[r, 3] * w_ref[3])
    y_ref[...] = jnp.dot(x_ref[...].astype(jnp.bfloat16),
                         rw.astype(jnp.bfloat16),
                         preferred_element_type=jnp.float32)

    @pl.when((i == 0) & (r == 0))
    def _():
        reo_ref[...] = lax.dot_general(
            re_ref[...], wrw_ref[...], (((1,), (1,)), ((), ())),
            preferred_element_type=jnp.float32) + wrb_ref[...]


def _sc_body(y_ref, idx_ref, acc0_ref, acc1_ref, deg0_ref, deg1_ref,
             ib0, ib1, rows0, rows1, ones_v, zflat_v, acc_sh, deg_sh,
             isem0, isem1, gsem0, gsem1):
    cid = lax.axis_index("c")
    sid = lax.axis_index("s")
    wid = cid * NSUB + sid
    base = sid * ROWS_PER_SUB
    sets = ((ib0, rows0, isem0, gsem0), (ib1, rows1, isem1, gsem1))

    zvec = jnp.zeros((16,), jnp.float32)
    onev = jnp.ones((16,), jnp.float32)
    for g in range(CB // 16):
        ones_v[pl.ds(g * 16, 16)] = onev

    def zfill(rr, _):
        for g in range(D // 16):
            rows0[rr, pl.ds(g * 16, 16)] = zvec
        return 0

    lax.fori_loop(0, CB, zfill, 0)

    def zflat(k, _):
        zflat_v[pl.ds(k * 16, 16)] = zvec
        return 0

    lax.fori_loop(0, ROWS_PER_SUB // 16, zflat, 0)

    for t in range(ROWS_PER_SUB // CB):
        pltpu.sync_copy(rows0, acc_sh.at[pl.ds(base + t * CB, CB)])
    pltpu.sync_copy(zflat_v, deg_sh.at[pl.ds(base, ROWS_PER_SUB)])
    plsc.subcore_barrier()

    def _idx_start(j, s):
        ib, _, isem, _ = sets[s]
        pltpu.make_async_copy(idx_ref.at[wid * CE + j], ib, isem).start()

    def _idx_wait(s):
        ib, _, isem, _ = sets[s]
        pltpu.make_async_copy(idx_ref.at[wid], ib, isem).wait()

    def _gather_start(s):
        ib, rows, _, gsem = sets[s]
        pltpu.make_async_copy(y_ref.at[ib.at[0]], rows, gsem).start()

    def _gather_wait_scatter(s):
        ib, rows, _, gsem = sets[s]
        pltpu.make_async_copy(y_ref.at[ib.at[0]], rows, gsem).wait()
        pltpu.sync_copy(rows, acc_sh.at[ib.at[1]], add=True)
        pltpu.sync_copy(ones_v, deg_sh.at[ib.at[1]], add=True)

    for t in range(NBUF):
        _idx_start(t, t)
    for t in range(NBUF - 1):
        _idx_wait(t)
        _gather_start(t)

    def body(k, _):
        for t in range(NBUF):
            j = k * NBUF + t
            s_next = (t + NBUF - 1) % NBUF

            @pl.when(j + NBUF - 1 < CE)
            def _():
                _idx_wait(s_next)
                _gather_start(s_next)

            _gather_wait_scatter(t)

            @pl.when(j + NBUF < CE)
            def _():
                _idx_start(j + NBUF, t)
        return 0

    lax.fori_loop(0, CE // NBUF, body, 0)

    plsc.subcore_barrier()

    @pl.when(cid == 0)
    def _():
        pltpu.sync_copy(acc_sh.at[pl.ds(base, ROWS_PER_SUB)],
                        acc0_ref.at[pl.ds(base, ROWS_PER_SUB)])
        pltpu.sync_copy(deg_sh.at[pl.ds(base, ROWS_PER_SUB)],
                        deg0_ref.at[pl.ds(base, ROWS_PER_SUB)])

    @pl.when(cid == 1)
    def _():
        pltpu.sync_copy(acc_sh.at[pl.ds(base, ROWS_PER_SUB)],
                        acc1_ref.at[pl.ds(base, ROWS_PER_SUB)])
        pltpu.sync_copy(deg_sh.at[pl.ds(base, ROWS_PER_SUB)],
                        deg1_ref.at[pl.ds(base, ROWS_PER_SUB)])


def _post_body(acc0_ref, acc1_ref, dcol_ref, x_ref, w_ref, out_ref):
    a = acc0_ref[...] + acc1_ref[...]
    alpha = 1.0 / jnp.maximum(dcol_ref[...], 1.0)
    xb = x_ref[...]
    h = a * alpha + jnp.dot(xb.astype(jnp.bfloat16),
                            w_ref[...].astype(jnp.bfloat16),
                            preferred_element_type=jnp.float32)
    out_ref[:, :D] = xb
    out_ref[:, D:] = h


def kernel(x, edge_index, edge_type, rel_emb, weight, w_comp,
           self_loop_weight, W_R_w, W_R_b):
    src = edge_index[0]
    dst = edge_index[1]
    gidx = edge_type.astype(jnp.int32) * N + src
    gidx_p = jnp.pad(gidx, (0, E_PAD - E)).reshape(NW * CE, 1, CB)
    dst_p = jnp.pad(dst, (0, E_PAD - E),
                    constant_values=N).reshape(NW * CE, 1, CB)
    idx2 = jnp.concatenate([gidx_p, dst_p], axis=1)

    y, rel_emb_new = pl.pallas_call(
        _pre_body,
        grid=(NBLK, R2),
        in_specs=[
            pl.BlockSpec(memory_space=pltpu.SMEM),
            pl.BlockSpec((NB, D, D), lambda i, r: (0, 0, 0)),
            pl.BlockSpec((TB, D), lambda i, r: (i, 0)),
            pl.BlockSpec((R2, D), lambda i, r: (0, 0)),
            pl.BlockSpec((D, D), lambda i, r: (0, 0)),
            pl.BlockSpec((1, D), lambda i, r: (0, 0)),
        ],
        out_specs=[
            pl.BlockSpec((TB, D), lambda i, r: (r * NBLK + i, 0)),
            pl.BlockSpec((R2, D), lambda i, r: (0, 0)),
        ],
        out_shape=[
            jax.ShapeDtypeStruct((R2 * N, D), jnp.float32),
            jax.ShapeDtypeStruct((R2, D), jnp.float32),
        ],
    )(w_comp, weight, x, rel_emb, W_R_w, W_R_b.reshape(1, D))

    mesh = plsc.VectorSubcoreMesh(core_axis_name="c", subcore_axis_name="s",
                                  num_cores=NCORES, num_subcores=NSUB)
    acc0, acc1, deg0, deg1 = pl.kernel(
        _sc_body,
        out_type=[jax.ShapeDtypeStruct((N_PAD, D), jnp.float32),
                  jax.ShapeDtypeStruct((N_PAD, D), jnp.float32),
                  jax.ShapeDtypeStruct((N_PAD,), jnp.float32),
                  jax.ShapeDtypeStruct((N_PAD,), jnp.float32)],
        mesh=mesh,
        compiler_params=pltpu.CompilerParams(use_tc_tiling_on_sc=False),
        scratch_types=(
            [pltpu.VMEM((2, CB), jnp.int32) for _ in range(NBUF)]
            + [pltpu.VMEM((CB, D), jnp.float32) for _ in range(NBUF)]
            + [pltpu.VMEM((CB,), jnp.float32),
               pltpu.VMEM((ROWS_PER_SUB,), jnp.float32),
               pltpu.VMEM_SHARED((N_PAD, D), jnp.float32),
               pltpu.VMEM_SHARED((N_PAD,), jnp.float32)]
            + [pltpu.SemaphoreType.DMA for _ in range(2 * NBUF)]
        ),
    )(y, idx2)

    dcol = (deg0 + deg1).reshape(N_PAD, 1)

    repr_ = pl.pallas_call(
        _post_body,
        grid=(NBLK,),
        in_specs=[
            pl.BlockSpec((TB, D), lambda i: (i, 0)),
            pl.BlockSpec((TB, D), lambda i: (i, 0)),
            pl.BlockSpec((TB, 1), lambda i: (i, 0)),
            pl.BlockSpec((TB, D), lambda i: (i, 0)),
            pl.BlockSpec((D, D), lambda i: (0, 0)),
        ],
        out_specs=pl.BlockSpec((TB, 2 * D), lambda i: (i, 0)),
        out_shape=jax.ShapeDtypeStruct((N, 2 * D), jnp.float32),
    )(acc0, acc1, dcol, x, self_loop_weight)

    return rel_emb_new, repr_
